# baseline (speedup 1.0000x reference)
import functools

import jax
import jax.numpy as jnp
from jax.experimental import pallas as pl
from jax.experimental.pallas import tpu as pltpu

_NUM_CLASSES = 19
_KK = 9
_EPS = 1e-12
_TH = 128


def _ns_kernel(nh, logits_ref, edges_ref, rtab_ref, out_ref):
    h = pl.program_id(1)
    th = _TH
    w = logits_ref.shape[3]

    m = logits_ref[0, 0]
    idx = jnp.zeros((th, w), dtype=jnp.int32)
    for c in range(1, _NUM_CLASSES):
        xc = logits_ref[0, c]
        gt = xc > m
        idx = jnp.where(gt, c, idx)
        m = jnp.where(gt, xc, m)
    idxb = idx.astype(jnp.bfloat16)

    e_cnt = edges_ref.shape[2]
    ei_top = jnp.maximum(2 * h - 2, 0)
    ei_bot = jnp.minimum(2 * h + 1, e_cnt - 1)
    et = edges_ref[0, :, pl.ds(ei_top, 1), :]
    eb = edges_ref[0, :, pl.ds(ei_bot, 1), :]
    mt = et[0]
    mb = eb[0]
    it_ = jnp.zeros((1, w), dtype=jnp.int32)
    ib_ = jnp.zeros((1, w), dtype=jnp.int32)
    for c in range(1, _NUM_CLASSES):
        gtt = et[c] > mt
        it_ = jnp.where(gtt, c, it_)
        mt = jnp.where(gtt, et[c], mt)
        gtb = eb[c] > mb
        ib_ = jnp.where(gtb, c, ib_)
        mb = jnp.where(gtb, eb[c], mb)
    neg = jnp.full((1, w), -1, dtype=jnp.int32)
    it_ = jnp.where(h > 0, it_, neg).astype(jnp.bfloat16)
    ib_ = jnp.where(h < nh - 1, ib_, neg).astype(jnp.bfloat16)

    ybig = jnp.concatenate([it_, idxb, ib_], axis=0)

    negcol = jnp.full((th + 2, 1), -1, dtype=jnp.bfloat16)
    yl = jnp.concatenate([ybig[:, 1:], negcol], axis=1)
    yr = jnp.concatenate([negcol, ybig[:, :-1]], axis=1)

    one = jnp.ones((th, w), dtype=jnp.bfloat16)
    zero = jnp.zeros((th, w), dtype=jnp.bfloat16)
    cnt = jnp.zeros((th, w), dtype=jnp.bfloat16)
    for dr in range(3):
        for y_sh, is_center_col in ((yl, False), (ybig, True), (yr, False)):
            if dr == 1 and is_center_col:
                continue
            cnt = cnt + jnp.where(y_sh[dr:dr + th, :] == idxb, one, zero)

    f = (idxb * jnp.bfloat16(_KK) + cnt).astype(jnp.int32)
    tab_lo = jnp.broadcast_to(rtab_ref[0:1, :], (th, 128))
    tab_hi = jnp.broadcast_to(rtab_ref[1:2, :], (th, 128))
    fhi = jnp.maximum(f - 128, 0)
    flo = jnp.minimum(f, 127)
    t0 = jnp.take_along_axis(tab_lo, flo, axis=1)
    t1 = jnp.take_along_axis(tab_hi, fhi, axis=1)
    recipf = jnp.where(f < 128, t0, t1)

    for c in range(_NUM_CLASSES):
        out_ref[0, c] = logits_ref[0, c] * recipf


@jax.jit
def kernel(logits, class_wise_nt):
    b, c, hgt, wid = logits.shape
    nh = hgt // _TH
    rflat = (1.0 / (jnp.maximum(class_wise_nt, 0.0) + _EPS)).reshape(-1)
    rtab = jnp.pad(rflat, (0, 256 - rflat.shape[0])).reshape(2, 128)

    rows = []
    for i in range(nh - 1):
        rows += [(i + 1) * _TH - 1, (i + 1) * _TH]
    edges = jnp.take(logits, jnp.asarray(rows, dtype=jnp.int32), axis=2)

    out = pl.pallas_call(
        functools.partial(_ns_kernel, nh),
        grid=(b, nh),
        in_specs=[
            pl.BlockSpec((1, c, _TH, wid), lambda bb, hh: (bb, 0, hh, 0)),
            pl.BlockSpec((1, c, 2 * (nh - 1), wid), lambda bb, hh: (bb, 0, 0, 0)),
            pl.BlockSpec((2, 128), lambda bb, hh: (0, 0)),
        ],
        out_specs=pl.BlockSpec((1, c, _TH, wid), lambda bb, hh: (bb, 0, hh, 0)),
        out_shape=jax.ShapeDtypeStruct(logits.shape, logits.dtype),
        compiler_params=pltpu.CompilerParams(
            dimension_semantics=("parallel", "parallel")),
    )(logits, edges, rtab)
    return out


# speedup vs baseline: 1.0353x; 1.0353x over previous
import functools

import jax
import jax.numpy as jnp
from jax.experimental import pallas as pl
from jax.experimental.pallas import tpu as pltpu

_NUM_CLASSES = 19
_KK = 9
_EPS = 1e-12
_TH = 256


def _ns_kernel(nh, logits_ref, edges_ref, rtab_ref, out_ref):
    h = pl.program_id(1)
    th = _TH
    w = logits_ref.shape[3]

    m = logits_ref[0, 0]
    idx = jnp.zeros((th, w), dtype=jnp.int32)
    for c in range(1, _NUM_CLASSES):
        xc = logits_ref[0, c]
        gt = xc > m
        idx = jnp.where(gt, c, idx)
        m = jnp.where(gt, xc, m)
    idxb = idx.astype(jnp.bfloat16)

    e_cnt = edges_ref.shape[2]
    ei_top = jnp.maximum(2 * h - 2, 0)
    ei_bot = jnp.minimum(2 * h + 1, e_cnt - 1)
    et = edges_ref[0, :, pl.ds(ei_top, 1), :]
    eb = edges_ref[0, :, pl.ds(ei_bot, 1), :]
    mt = et[0]
    mb = eb[0]
    it_ = jnp.zeros((1, w), dtype=jnp.int32)
    ib_ = jnp.zeros((1, w), dtype=jnp.int32)
    for c in range(1, _NUM_CLASSES):
        gtt = et[c] > mt
        it_ = jnp.where(gtt, c, it_)
        mt = jnp.where(gtt, et[c], mt)
        gtb = eb[c] > mb
        ib_ = jnp.where(gtb, c, ib_)
        mb = jnp.where(gtb, eb[c], mb)
    neg = jnp.full((1, w), -1, dtype=jnp.int32)
    it_ = jnp.where(h > 0, it_, neg).astype(jnp.bfloat16)
    ib_ = jnp.where(h < nh - 1, ib_, neg).astype(jnp.bfloat16)

    ybig = jnp.concatenate([it_, idxb, ib_], axis=0)

    negcol = jnp.full((th + 2, 1), -1, dtype=jnp.bfloat16)
    yl = jnp.concatenate([ybig[:, 1:], negcol], axis=1)
    yr = jnp.concatenate([negcol, ybig[:, :-1]], axis=1)

    one = jnp.ones((th, w), dtype=jnp.bfloat16)
    zero = jnp.zeros((th, w), dtype=jnp.bfloat16)
    cnt = jnp.zeros((th, w), dtype=jnp.bfloat16)
    for dr in range(3):
        for y_sh, is_center_col in ((yl, False), (ybig, True), (yr, False)):
            if dr == 1 and is_center_col:
                continue
            cnt = cnt + jnp.where(y_sh[dr:dr + th, :] == idxb, one, zero)

    f = (idxb * jnp.bfloat16(_KK) + cnt).astype(jnp.int32)
    tab_lo = jnp.broadcast_to(rtab_ref[0:1, :], (th, 128))
    tab_hi = jnp.broadcast_to(rtab_ref[1:2, :], (th, 128))
    fhi = jnp.maximum(f - 128, 0)
    flo = jnp.minimum(f, 127)
    t0 = jnp.take_along_axis(tab_lo, flo, axis=1)
    t1 = jnp.take_along_axis(tab_hi, fhi, axis=1)
    recipf = jnp.where(f < 128, t0, t1)

    for c in range(_NUM_CLASSES):
        out_ref[0, c] = logits_ref[0, c] * recipf


@jax.jit
def kernel(logits, class_wise_nt):
    b, c, hgt, wid = logits.shape
    nh = hgt // _TH
    rflat = (1.0 / (jnp.maximum(class_wise_nt, 0.0) + _EPS)).reshape(-1)
    rtab = jnp.pad(rflat, (0, 256 - rflat.shape[0])).reshape(2, 128)

    rows = []
    for i in range(nh - 1):
        rows += [(i + 1) * _TH - 1, (i + 1) * _TH]
    edges = jnp.take(logits, jnp.asarray(rows, dtype=jnp.int32), axis=2)

    out = pl.pallas_call(
        functools.partial(_ns_kernel, nh),
        grid=(b, nh),
        in_specs=[
            pl.BlockSpec((1, c, _TH, wid), lambda bb, hh: (bb, 0, hh, 0)),
            pl.BlockSpec((1, c, 2 * (nh - 1), wid), lambda bb, hh: (bb, 0, 0, 0)),
            pl.BlockSpec((2, 128), lambda bb, hh: (0, 0)),
        ],
        out_specs=pl.BlockSpec((1, c, _TH, wid), lambda bb, hh: (bb, 0, hh, 0)),
        out_shape=jax.ShapeDtypeStruct(logits.shape, logits.dtype),
        compiler_params=pltpu.CompilerParams(
            dimension_semantics=("arbitrary", "arbitrary")),
    )(logits, edges, rtab)
    return out


# speedup vs baseline: 1.0403x; 1.0048x over previous
import functools

import jax
import jax.numpy as jnp
from jax.experimental import pallas as pl
from jax.experimental.pallas import tpu as pltpu

_NUM_CLASSES = 19
_KK = 9
_EPS = 1e-12
_TH = 256


def _ns_kernel(nh, logits_ref, edges_ref, rtab_ref, out_ref):
    h = pl.program_id(1)
    th = _TH
    w = logits_ref.shape[3]

    m = logits_ref[0, 0]
    idx = jnp.zeros((th, w), dtype=jnp.int32)
    for c in range(1, _NUM_CLASSES):
        xc = logits_ref[0, c]
        gt = xc > m
        idx = jnp.where(gt, c, idx)
        m = jnp.where(gt, xc, m)
    idxb = idx.astype(jnp.bfloat16)

    e_cnt = edges_ref.shape[2]
    ei_top = jnp.maximum(2 * h - 2, 0)
    ei_bot = jnp.minimum(2 * h + 1, e_cnt - 1)
    et = edges_ref[0, :, pl.ds(ei_top, 1), :]
    eb = edges_ref[0, :, pl.ds(ei_bot, 1), :]
    mt = et[0]
    mb = eb[0]
    it_ = jnp.zeros((1, w), dtype=jnp.int32)
    ib_ = jnp.zeros((1, w), dtype=jnp.int32)
    for c in range(1, _NUM_CLASSES):
        gtt = et[c] > mt
        it_ = jnp.where(gtt, c, it_)
        mt = jnp.where(gtt, et[c], mt)
        gtb = eb[c] > mb
        ib_ = jnp.where(gtb, c, ib_)
        mb = jnp.where(gtb, eb[c], mb)
    neg = jnp.full((1, w), -1, dtype=jnp.int32)
    it_ = jnp.where(h > 0, it_, neg).astype(jnp.bfloat16)
    ib_ = jnp.where(h < nh - 1, ib_, neg).astype(jnp.bfloat16)

    ybig = jnp.concatenate([it_, idxb, ib_], axis=0)

    negcol = jnp.full((th + 2, 1), -1, dtype=jnp.bfloat16)
    yl = jnp.concatenate([ybig[:, 1:], negcol], axis=1)
    yr = jnp.concatenate([negcol, ybig[:, :-1]], axis=1)

    one = jnp.ones((th, w), dtype=jnp.bfloat16)
    zero = jnp.zeros((th, w), dtype=jnp.bfloat16)
    cnt = jnp.zeros((th, w), dtype=jnp.bfloat16)
    for dr in range(3):
        for y_sh, is_center_col in ((yl, False), (ybig, True), (yr, False)):
            if dr == 1 and is_center_col:
                continue
            cnt = cnt + jnp.where(y_sh[dr:dr + th, :] == idxb, one, zero)

    f = (idxb * jnp.bfloat16(_KK) + cnt).astype(jnp.int32)
    tab_lo = jnp.broadcast_to(rtab_ref[0:1, :], (th, 128))
    tab_hi = jnp.broadcast_to(rtab_ref[1:2, :], (th, 128))
    fhi = jnp.maximum(f - 128, 0)
    flo = jnp.minimum(f, 127)
    t0 = jnp.take_along_axis(tab_lo, flo, axis=1)
    t1 = jnp.take_along_axis(tab_hi, fhi, axis=1)
    recipf = jnp.where(f < 128, t0, t1)

    for c in range(_NUM_CLASSES):
        out_ref[0, c] = logits_ref[0, c] * recipf


@jax.jit
def kernel(logits, class_wise_nt):
    b, c, hgt, wid = logits.shape
    nh = hgt // _TH
    rflat = (1.0 / (jnp.maximum(class_wise_nt, 0.0) + _EPS)).reshape(-1)
    rtab = jnp.pad(rflat, (0, 256 - rflat.shape[0])).reshape(2, 128)

    edges = jnp.concatenate(
        [logits[:, :, (i + 1) * _TH - 1:(i + 1) * _TH + 1, :]
         for i in range(nh - 1)], axis=2) if nh > 2 else (
        logits[:, :, _TH - 1:_TH + 1, :])

    out = pl.pallas_call(
        functools.partial(_ns_kernel, nh),
        grid=(b, nh),
        in_specs=[
            pl.BlockSpec((1, c, _TH, wid), lambda bb, hh: (bb, 0, hh, 0)),
            pl.BlockSpec((1, c, 2 * (nh - 1), wid), lambda bb, hh: (bb, 0, 0, 0)),
            pl.BlockSpec((2, 128), lambda bb, hh: (0, 0)),
        ],
        out_specs=pl.BlockSpec((1, c, _TH, wid), lambda bb, hh: (bb, 0, hh, 0)),
        out_shape=jax.ShapeDtypeStruct(logits.shape, logits.dtype),
        compiler_params=pltpu.CompilerParams(
            dimension_semantics=("arbitrary", "arbitrary")),
    )(logits, edges, rtab)
    return out
